# baseline (speedup 1.0000x reference)
import functools

import jax
import jax.numpy as jnp
from jax import lax
from jax.experimental import pallas as pl
from jax.experimental.pallas import tpu as pltpu
from jax.experimental.pallas import tpu_sc as plsc

N_NODES_ = 10000
N_EDGES_ = 320000
N_CH_ = 128

NC = 2
NS = 16
NW = NC * NS
EPW = N_EDGES_ // NW
C = 80
NCHUNK = EPW // C


def _normalize_rows_tc(x):
    def body(x_ref, o_ref):
        v = x_ref[...]
        o_ref[...] = v * lax.rsqrt(jnp.sum(v * v, axis=1, keepdims=True))

    return pl.pallas_call(
        body,
        out_shape=jax.ShapeDtypeStruct(x.shape, x.dtype),
    )(x)


@functools.partial(
    pl.kernel,
    out_type=jax.ShapeDtypeStruct((N_EDGES_,), jnp.float32),
    mesh=plsc.VectorSubcoreMesh(core_axis_name="c", subcore_axis_name="s"),
    compiler_params=pltpu.CompilerParams(needs_layout_passes=False),
    scratch_types=dict(
        idx_s=pltpu.VMEM((C,), jnp.int32),
        idx_d=pltpu.VMEM((C,), jnp.int32),
        idx_r=pltpu.VMEM((C,), jnp.int32),
        s_rows=[pltpu.VMEM((C, N_CH_), jnp.float32) for _ in range(1)],
        o_rows=[pltpu.VMEM((C, N_CH_), jnp.float32) for _ in range(1)],
        r_rows=[pltpu.VMEM((C, N_CH_), jnp.float32) for _ in range(1)],
        out_v=pltpu.VMEM((C,), jnp.float32),
        sem_s=[pltpu.SemaphoreType.DMA for _ in range(1)],
        sem_o=[pltpu.SemaphoreType.DMA for _ in range(1)],
        sem_r=[pltpu.SemaphoreType.DMA for _ in range(1)],
        xn_sp=pltpu.VMEM_SHARED((N_NODES_, N_CH_), jnp.float32),
        w_sp=pltpu.VMEM_SHARED((500, N_CH_), jnp.float32),
    ),
)
def _distmult_sc(xn_hbm, src_hbm, dst_hbm, rel_hbm, w_hbm, out_hbm,
                 idx_s, idx_d, idx_r, s_rows, o_rows, r_rows, out_v,
                 sem_s, sem_o, sem_r, xn_sp, w_sp):
    sid = lax.axis_index("s")
    wid = sid * NC + lax.axis_index("c")
    base = wid * EPW
    @pl.when(sid == 0)
    def _():
        pltpu.sync_copy(xn_hbm, xn_sp)
        pltpu.sync_copy(w_hbm, w_sp)

    plsc.subcore_barrier()

    lane = lax.iota(jnp.int32, 16)

    def fire(ci, b):
        off = base + ci * C
        pltpu.sync_copy(src_hbm.at[pl.ds(off, C)], idx_s)
        pltpu.sync_copy(dst_hbm.at[pl.ds(off, C)], idx_d)
        pltpu.sync_copy(rel_hbm.at[pl.ds(off, C)], idx_r)
        pltpu.async_copy(xn_sp.at[idx_s], s_rows[b], sem_s[b])
        pltpu.async_copy(xn_sp.at[idx_d], o_rows[b], sem_o[b])
        pltpu.async_copy(w_sp.at[idx_r], r_rows[b], sem_r[b])

    def drain(ci, b):
        pltpu.make_async_copy(xn_sp.at[idx_s], s_rows[b], sem_s[b]).wait()
        pltpu.make_async_copy(xn_sp.at[idx_d], o_rows[b], sem_o[b]).wait()
        pltpu.make_async_copy(w_sp.at[idx_r], r_rows[b], sem_r[b]).wait()

    def compute(ci, b):
        off = ci * C
        sb, rb, ob = s_rows[b], r_rows[b], o_rows[b]

        def group_body(g, c2):
            vec = jnp.zeros((16,), jnp.float32)
            for m in range(16):
                e = g * 16 + m
                acc = sb[e, pl.ds(0, 16)] * rb[e, pl.ds(0, 16)] * ob[e, pl.ds(0, 16)]
                for k in range(1, N_CH_ // 16):
                    acc = acc + (sb[e, pl.ds(k * 16, 16)]
                                 * rb[e, pl.ds(k * 16, 16)]
                                 * ob[e, pl.ds(k * 16, 16)])
                vec = jnp.where(lane == m, jnp.sum(acc), vec)
            out_v[pl.ds(g * 16, 16)] = vec
            return c2

        lax.fori_loop(0, C // 16, group_body, 0)

    def chunk_body(i, carry):
        fire(i, 0)
        drain(i, 0)
        compute(i, 0)
        pltpu.sync_copy(out_v, out_hbm.at[pl.ds(base + i * C, C)])
        return carry

    lax.fori_loop(0, NCHUNK, chunk_body, 0)


def kernel(x, edge_index, edge_type, weights):
    xn = _normalize_rows_tc(x)
    src = edge_index[0, :].astype(jnp.int32)
    dst = edge_index[1, :].astype(jnp.int32)
    rel = edge_type.astype(jnp.int32)
    return _distmult_sc(xn, src, dst, rel, weights)


# speedup vs baseline: 1.3961x; 1.3961x over previous
import functools

import jax
import jax.numpy as jnp
from jax import lax
from jax.experimental import pallas as pl
from jax.experimental.pallas import tpu as pltpu
from jax.experimental.pallas import tpu_sc as plsc

N_NODES_ = 10000
N_EDGES_ = 320000
N_CH_ = 128

NC = 2
NS = 16
NW = NC * NS
EPW = N_EDGES_ // NW
C = 80
NCHUNK = EPW // C


def _normalize_rows_tc(x):
    def body(x_ref, o_ref):
        v = x_ref[...]
        o_ref[...] = v * lax.rsqrt(jnp.sum(v * v, axis=1, keepdims=True))

    return pl.pallas_call(
        body,
        out_shape=jax.ShapeDtypeStruct(x.shape, x.dtype),
    )(x)


@functools.partial(
    pl.kernel,
    out_type=jax.ShapeDtypeStruct((N_EDGES_,), jnp.float32),
    mesh=plsc.VectorSubcoreMesh(core_axis_name="c", subcore_axis_name="s"),
    compiler_params=pltpu.CompilerParams(needs_layout_passes=False),
    scratch_types=dict(
        idx_s=pltpu.VMEM((EPW,), jnp.int32),
        idx_d=pltpu.VMEM((EPW,), jnp.int32),
        idx_r=pltpu.VMEM((EPW,), jnp.int32),
        s_rows=[pltpu.VMEM((C, N_CH_), jnp.float32) for _ in range(2)],
        o_rows=[pltpu.VMEM((C, N_CH_), jnp.float32) for _ in range(2)],
        r_rows=[pltpu.VMEM((C, N_CH_), jnp.float32) for _ in range(2)],
        out_v=pltpu.VMEM((EPW,), jnp.float32),
        sem_s=[pltpu.SemaphoreType.DMA for _ in range(2)],
        sem_o=[pltpu.SemaphoreType.DMA for _ in range(2)],
        sem_r=[pltpu.SemaphoreType.DMA for _ in range(2)],
    ),
)
def _distmult_sc(xn_hbm, src_hbm, dst_hbm, rel_hbm, w_hbm, out_hbm,
                 idx_s, idx_d, idx_r, s_rows, o_rows, r_rows, out_v,
                 sem_s, sem_o, sem_r):
    wid = lax.axis_index("s") * NC + lax.axis_index("c")
    base = wid * EPW
    pltpu.sync_copy(src_hbm.at[pl.ds(base, EPW)], idx_s)
    pltpu.sync_copy(dst_hbm.at[pl.ds(base, EPW)], idx_d)
    pltpu.sync_copy(rel_hbm.at[pl.ds(base, EPW)], idx_r)

    lane = lax.iota(jnp.int32, 16)

    def fire(ci, b):
        off = ci * C
        pltpu.async_copy(xn_hbm.at[idx_s.at[pl.ds(off, C)]], s_rows[b], sem_s[b])
        pltpu.async_copy(xn_hbm.at[idx_d.at[pl.ds(off, C)]], o_rows[b], sem_o[b])
        pltpu.async_copy(w_hbm.at[idx_r.at[pl.ds(off, C)]], r_rows[b], sem_r[b])

    def drain(ci, b):
        off = ci * C
        pltpu.make_async_copy(
            xn_hbm.at[idx_s.at[pl.ds(off, C)]], s_rows[b], sem_s[b]).wait()
        pltpu.make_async_copy(
            xn_hbm.at[idx_d.at[pl.ds(off, C)]], o_rows[b], sem_o[b]).wait()
        pltpu.make_async_copy(
            w_hbm.at[idx_r.at[pl.ds(off, C)]], r_rows[b], sem_r[b]).wait()

    def compute(ci, b):
        off = ci * C
        sb, rb, ob = s_rows[b], r_rows[b], o_rows[b]

        def group_body(g, c2):
            vec = jnp.zeros((16,), jnp.float32)
            for m in range(16):
                e = g * 16 + m
                acc = sb[e, pl.ds(0, 16)] * rb[e, pl.ds(0, 16)] * ob[e, pl.ds(0, 16)]
                for k in range(1, N_CH_ // 16):
                    acc = acc + (sb[e, pl.ds(k * 16, 16)]
                                 * rb[e, pl.ds(k * 16, 16)]
                                 * ob[e, pl.ds(k * 16, 16)])
                vec = jnp.where(lane == m, jnp.sum(acc), vec)
            out_v[pl.ds(off + g * 16, 16)] = vec
            return c2

        lax.fori_loop(0, C // 16, group_body, 0)

    fire(0, 0)

    def pair_body(i, carry):
        c0 = 2 * i
        fire(c0 + 1, 1)
        drain(c0, 0)
        compute(c0, 0)
        fire(c0 + 2, 0)
        drain(c0 + 1, 1)
        compute(c0 + 1, 1)
        return carry

    lax.fori_loop(0, (NCHUNK - 1) // 2, pair_body, 0)
    drain(NCHUNK - 1, 0)
    compute(NCHUNK - 1, 0)

    pltpu.sync_copy(out_v, out_hbm.at[pl.ds(base, EPW)])


def kernel(x, edge_index, edge_type, weights):
    xn = _normalize_rows_tc(x)
    src = edge_index[0, :].astype(jnp.int32)
    dst = edge_index[1, :].astype(jnp.int32)
    rel = edge_type.astype(jnp.int32)
    return _distmult_sc(xn, src, dst, rel, weights)


# speedup vs baseline: 4.7798x; 3.4237x over previous
import functools

import jax
import jax.numpy as jnp
from jax import lax
from jax.experimental import pallas as pl
from jax.experimental.pallas import tpu as pltpu
from jax.experimental.pallas import tpu_sc as plsc

N_NODES_ = 10000
N_EDGES_ = 320000
N_CH_ = 128

NC = 2
NS = 16
NW = NC * NS
EPW = N_EDGES_ // NW
C = 80
NCHUNK = EPW // C


def _normalize_rows_tc(x):
    def body(x_ref, o_ref):
        v = x_ref[...]
        o_ref[...] = (v * lax.rsqrt(jnp.sum(v * v, axis=1, keepdims=True))
                      ).astype(jnp.bfloat16)

    return pl.pallas_call(
        body,
        out_shape=jax.ShapeDtypeStruct(x.shape, jnp.bfloat16),
    )(x)


@functools.partial(
    pl.kernel,
    out_type=jax.ShapeDtypeStruct((N_EDGES_,), jnp.float32),
    mesh=plsc.VectorSubcoreMesh(core_axis_name="c", subcore_axis_name="s"),
    compiler_params=pltpu.CompilerParams(
        needs_layout_passes=False, use_tc_tiling_on_sc=False),
    scratch_types=dict(
        idx_s=pltpu.VMEM((EPW,), jnp.int32),
        idx_d=pltpu.VMEM((EPW,), jnp.int32),
        idx_r=pltpu.VMEM((EPW,), jnp.int32),
        s_rows=[pltpu.VMEM((C, N_CH_ // 2), jnp.int32) for _ in range(2)],
        o_rows=[pltpu.VMEM((C, N_CH_ // 2), jnp.int32) for _ in range(2)],
        r_rows=[pltpu.VMEM((C, N_CH_ // 2), jnp.int32) for _ in range(2)],
        out_v=pltpu.VMEM((EPW,), jnp.float32),
        sem_s=[pltpu.SemaphoreType.DMA for _ in range(2)],
        sem_o=[pltpu.SemaphoreType.DMA for _ in range(2)],
        sem_r=[pltpu.SemaphoreType.DMA for _ in range(2)],
    ),
)
def _distmult_sc(xn_hbm, src_hbm, dst_hbm, rel_hbm, w_hbm, out_hbm,
                 idx_s, idx_d, idx_r, s_rows, o_rows, r_rows, out_v,
                 sem_s, sem_o, sem_r):
    wid = lax.axis_index("s") * NC + lax.axis_index("c")
    base = wid * EPW
    pltpu.sync_copy(src_hbm.at[pl.ds(base, EPW)], idx_s)
    pltpu.sync_copy(dst_hbm.at[pl.ds(base, EPW)], idx_d)
    pltpu.sync_copy(rel_hbm.at[pl.ds(base, EPW)], idx_r)

    lane = lax.iota(jnp.int32, 16)

    def fire(ci, b):
        off = ci * C
        pltpu.async_copy(xn_hbm.at[idx_s.at[pl.ds(off, C)]], s_rows[b], sem_s[b])
        pltpu.async_copy(xn_hbm.at[idx_d.at[pl.ds(off, C)]], o_rows[b], sem_o[b])
        pltpu.async_copy(w_hbm.at[idx_r.at[pl.ds(off, C)]], r_rows[b], sem_r[b])

    def drain(ci, b):
        off = ci * C
        pltpu.make_async_copy(
            xn_hbm.at[idx_s.at[pl.ds(off, C)]], s_rows[b], sem_s[b]).wait()
        pltpu.make_async_copy(
            xn_hbm.at[idx_d.at[pl.ds(off, C)]], o_rows[b], sem_o[b]).wait()
        pltpu.make_async_copy(
            w_hbm.at[idx_r.at[pl.ds(off, C)]], r_rows[b], sem_r[b]).wait()

    def compute(ci, b):
        off = ci * C
        sb, rb, ob = s_rows[b], r_rows[b], o_rows[b]

        def group_body(g, c2):
            vec = jnp.zeros((16,), jnp.float32)
            for m in range(16):
                e = g * 16 + m
                acc = jnp.zeros((16,), jnp.float32)
                for k in range(N_CH_ // 32):
                    sw = plsc.bitcast(sb[e, pl.ds(k * 16, 16)], jnp.bfloat16)
                    rw = plsc.bitcast(rb[e, pl.ds(k * 16, 16)], jnp.bfloat16)
                    ow = plsc.bitcast(ob[e, pl.ds(k * 16, 16)], jnp.bfloat16)
                    sa, sb2 = plsc.unpack(sw, format=plsc.PackFormat.INTERLEAVED)
                    ra, rb2 = plsc.unpack(rw, format=plsc.PackFormat.INTERLEAVED)
                    oa, ob2 = plsc.unpack(ow, format=plsc.PackFormat.INTERLEAVED)
                    acc = acc + sa * ra * oa + sb2 * rb2 * ob2
                vec = jnp.where(lane == m, jnp.sum(acc), vec)
            out_v[pl.ds(off + g * 16, 16)] = vec
            return c2

        lax.fori_loop(0, C // 16, group_body, 0)

    fire(0, 0)

    def pair_body(i, carry):
        c0 = 2 * i
        fire(c0 + 1, 1)
        drain(c0, 0)
        compute(c0, 0)
        fire(c0 + 2, 0)
        drain(c0 + 1, 1)
        compute(c0 + 1, 1)
        return carry

    lax.fori_loop(0, (NCHUNK - 1) // 2, pair_body, 0)
    drain(NCHUNK - 1, 0)
    compute(NCHUNK - 1, 0)

    pltpu.sync_copy(out_v, out_hbm.at[pl.ds(base, EPW)])


def _as_i32_pairs(a_bf16):
    n, c = a_bf16.shape
    return lax.bitcast_convert_type(
        a_bf16.reshape(n, c // 2, 2), jnp.int32)


def kernel(x, edge_index, edge_type, weights):
    xn = _as_i32_pairs(_normalize_rows_tc(x))
    src = edge_index[0, :].astype(jnp.int32)
    dst = edge_index[1, :].astype(jnp.int32)
    rel = edge_type.astype(jnp.int32)
    w = _as_i32_pairs(weights.astype(jnp.bfloat16))
    return _distmult_sc(xn, src, dst, rel, w)


# speedup vs baseline: 5.6102x; 1.1737x over previous
import functools

import jax
import jax.numpy as jnp
from jax import lax
from jax.experimental import pallas as pl
from jax.experimental.pallas import tpu as pltpu
from jax.experimental.pallas import tpu_sc as plsc

N_NODES_ = 10000
N_EDGES_ = 320000
N_CH_ = 128

NC = 2
NS = 16
NW = NC * NS
EPW = N_EDGES_ // NW
C = 80
NCHUNK = EPW // C


def _normalize_rows_tc(x):
    def body(x_ref, o_ref):
        v = x_ref[...]
        o_ref[...] = (v * lax.rsqrt(jnp.sum(v * v, axis=1, keepdims=True))
                      ).astype(jnp.bfloat16)

    return pl.pallas_call(
        body,
        out_shape=jax.ShapeDtypeStruct(x.shape, jnp.bfloat16),
    )(x)


@functools.partial(
    pl.kernel,
    out_type=jax.ShapeDtypeStruct((N_EDGES_,), jnp.float32),
    mesh=plsc.VectorSubcoreMesh(core_axis_name="c", subcore_axis_name="s"),
    compiler_params=pltpu.CompilerParams(
        needs_layout_passes=False, use_tc_tiling_on_sc=False),
    scratch_types=dict(
        idx_s=pltpu.VMEM((EPW,), jnp.int32),
        idx_d=pltpu.VMEM((EPW,), jnp.int32),
        idx_r=pltpu.VMEM((EPW,), jnp.int32),
        s_rows=[pltpu.VMEM((C, N_CH_ // 2), jnp.int32) for _ in range(2)],
        o_rows=[pltpu.VMEM((C, N_CH_ // 2), jnp.int32) for _ in range(2)],
        r_rows=[pltpu.VMEM((C, N_CH_ // 2), jnp.int32) for _ in range(2)],
        out_v=pltpu.VMEM((EPW,), jnp.float32),
        sem_s=[pltpu.SemaphoreType.DMA for _ in range(2)],
        sem_o=[pltpu.SemaphoreType.DMA for _ in range(2)],
        sem_r=[pltpu.SemaphoreType.DMA for _ in range(2)],
        xn_sp=pltpu.VMEM_SHARED((N_NODES_, N_CH_ // 2), jnp.int32),
        w_sp=pltpu.VMEM_SHARED((500, N_CH_ // 2), jnp.int32),
    ),
)
def _distmult_sc(xn_hbm, src_hbm, dst_hbm, rel_hbm, w_hbm, out_hbm,
                 idx_s, idx_d, idx_r, s_rows, o_rows, r_rows, out_v,
                 sem_s, sem_o, sem_r, xn_sp, w_sp):
    sid = lax.axis_index("s")
    wid = sid * NC + lax.axis_index("c")
    base = wid * EPW
    @pl.when(sid == 0)
    def _():
        pltpu.sync_copy(xn_hbm, xn_sp)
        pltpu.sync_copy(w_hbm, w_sp)

    pltpu.sync_copy(src_hbm.at[pl.ds(base, EPW)], idx_s)
    pltpu.sync_copy(dst_hbm.at[pl.ds(base, EPW)], idx_d)
    pltpu.sync_copy(rel_hbm.at[pl.ds(base, EPW)], idx_r)
    plsc.subcore_barrier()

    lane = lax.iota(jnp.int32, 16)

    def fire(ci, b):
        off = ci * C
        pltpu.async_copy(xn_sp.at[idx_s.at[pl.ds(off, C)]], s_rows[b], sem_s[b])
        pltpu.async_copy(xn_sp.at[idx_d.at[pl.ds(off, C)]], o_rows[b], sem_o[b])
        pltpu.async_copy(w_sp.at[idx_r.at[pl.ds(off, C)]], r_rows[b], sem_r[b])

    def drain(ci, b):
        off = ci * C
        pltpu.make_async_copy(
            xn_sp.at[idx_s.at[pl.ds(off, C)]], s_rows[b], sem_s[b]).wait()
        pltpu.make_async_copy(
            xn_sp.at[idx_d.at[pl.ds(off, C)]], o_rows[b], sem_o[b]).wait()
        pltpu.make_async_copy(
            w_sp.at[idx_r.at[pl.ds(off, C)]], r_rows[b], sem_r[b]).wait()

    def compute(ci, b):
        off = ci * C
        sb, rb, ob = s_rows[b], r_rows[b], o_rows[b]

        def group_body(g, c2):
            vec = jnp.zeros((16,), jnp.float32)
            for m in range(16):
                e = g * 16 + m
                acc = jnp.zeros((16,), jnp.float32)
                for k in range(N_CH_ // 32):
                    sw = plsc.bitcast(sb[e, pl.ds(k * 16, 16)], jnp.bfloat16)
                    rw = plsc.bitcast(rb[e, pl.ds(k * 16, 16)], jnp.bfloat16)
                    ow = plsc.bitcast(ob[e, pl.ds(k * 16, 16)], jnp.bfloat16)
                    sa, sb2 = plsc.unpack(sw, format=plsc.PackFormat.INTERLEAVED)
                    ra, rb2 = plsc.unpack(rw, format=plsc.PackFormat.INTERLEAVED)
                    oa, ob2 = plsc.unpack(ow, format=plsc.PackFormat.INTERLEAVED)
                    acc = acc + sa * ra * oa + sb2 * rb2 * ob2
                vec = jnp.where(lane == m, jnp.sum(acc), vec)
            out_v[pl.ds(off + g * 16, 16)] = vec
            return c2

        lax.fori_loop(0, C // 16, group_body, 0)

    fire(0, 0)

    def pair_body(i, carry):
        c0 = 2 * i
        fire(c0 + 1, 1)
        drain(c0, 0)
        compute(c0, 0)
        fire(c0 + 2, 0)
        drain(c0 + 1, 1)
        compute(c0 + 1, 1)
        return carry

    lax.fori_loop(0, (NCHUNK - 1) // 2, pair_body, 0)
    drain(NCHUNK - 1, 0)
    compute(NCHUNK - 1, 0)

    pltpu.sync_copy(out_v, out_hbm.at[pl.ds(base, EPW)])


def _as_i32_pairs(a_bf16):
    n, c = a_bf16.shape
    return lax.bitcast_convert_type(
        a_bf16.reshape(n, c // 2, 2), jnp.int32)


def kernel(x, edge_index, edge_type, weights):
    xn = _as_i32_pairs(_normalize_rows_tc(x))
    src = edge_index[0, :].astype(jnp.int32)
    dst = edge_index[1, :].astype(jnp.int32)
    rel = edge_type.astype(jnp.int32)
    w = _as_i32_pairs(weights.astype(jnp.bfloat16))
    return _distmult_sc(xn, src, dst, rel, w)
